# baseline (device time: 81611 ns/iter reference)
import jax
import jax.numpy as jnp
from jax import lax
from jax.experimental import pallas as pl
from jax.experimental.pallas import tpu as pltpu

N_DEV = 8
M_OUT = 512
G_OFF = (0, 768, 1408)
G_COLS = (768, 640, 640)
G_DIMS = ((0, 1, 2), (1, 2, 0), (2, 0, 1))
OTHER = {0: (1, 2), 1: (0, 2), 2: (0, 1)}


def _ring(p):
    return jnp.where(p < 4, p, 11 - p)


def kernel(x, w_mat):
    m, k = x.shape
    _, n = w_mat.shape

    def body(x_ref, w_ref, out_ref, maxsrc_ref, maxbuf_ref,
             p0, p1, p2, rb1_0, rb1_1, rb1_2, rb2_0, rb2_1, rb2_2,
             rb3_0, rb3_1, rb3_2, ss0, ss1, ss2, rs0, rs1, rs2,
             msend_sems, mrecv_sems):
        d = lax.axis_index("i")
        m4 = d % 4
        mybits = [
            jnp.where((m4 == 1) | (m4 == 2), 1, 0),
            jnp.where(m4 >= 2, 1, 0),
            jnp.where(d >= 4, 1, 0),
        ]

        def cid(bits):
            return jnp.where(bits[1] == 0, bits[0], 3 - bits[0]) + 4 * bits[2]

        def flip(bits, dim):
            b = list(bits)
            b[dim] = 1 - b[dim]
            return b

        neighbors = [cid(flip(mybits, dim)) for dim in range(3)]

        barrier_sem = pltpu.get_barrier_semaphore()
        for nbr in neighbors:
            pl.semaphore_signal(
                barrier_sem, inc=1,
                device_id=(nbr,), device_id_type=pl.DeviceIdType.MESH,
            )
        pl.semaphore_wait(barrier_sem, 3)

        P = [p0, p1, p2]
        RB1 = [rb1_0, rb1_1, rb1_2]
        RB2 = [rb2_0, rb2_1, rb2_2]
        RB3 = [rb3_0, rb3_1, rb3_2]
        SS = [ss0, ss1, ss2]
        RS = [rs0, rs1, rs2]

        def r1_chunk(g, j, side_bit):
            d1, d2, d3 = G_DIMS[g]
            bits = [None, None, None]
            bits[d1] = side_bit
            bits[d2] = mybits[d2] ^ (1 if j < 2 else 0)
            bits[d3] = mybits[d3] ^ (j & 1)
            return cid(bits)

        def r2_chunk(g, j, side_bit):
            d1, d2, d3 = G_DIMS[g]
            bits = [None, None, None]
            bits[d1] = mybits[d1]
            bits[d2] = side_bit
            bits[d3] = mybits[d3] ^ (1 if j == 0 else 0)
            return cid(bits)

        def dot_chunk(g, ck):
            return jnp.dot(
                x_ref[pl.ds(ck * M_OUT, M_OUT), :],
                w_ref[:, G_OFF[g]:G_OFF[g] + G_COLS[g]],
                preferred_element_type=jnp.float32,
            )

        def rdma(g, sem_idx, src_row, dst_ref, dim):
            return pltpu.make_async_remote_copy(
                src_ref=P[g].at[pl.ds(src_row * M_OUT, M_OUT)],
                dst_ref=dst_ref,
                send_sem=SS[g].at[sem_idx],
                recv_sem=RS[g].at[sem_idx],
                device_id=(neighbors[dim],),
                device_id_type=pl.DeviceIdType.MESH,
            )

        send_descs = []

        r1_descs = [[None] * 4 for _ in range(3)]
        for j in range(4):
            for g in range(3):
                dim = G_DIMS[g][0]
                ck = r1_chunk(g, j, 1 - mybits[dim])
                P[g][pl.ds(ck * M_OUT, M_OUT)] = (
                    dot_chunk(g, ck).astype(jnp.bfloat16)
                )
                r = rdma(g, j, ck, RB1[g].at[j], dim)
                r.start()
                r1_descs[g][j] = r
                send_descs.append(r)

        for j in range(4):
            for g in range(3):
                dim = G_DIMS[g][0]
                ck = r1_chunk(g, j, mybits[dim])
                P[g][pl.ds(ck * M_OUT, M_OUT)] = (
                    dot_chunk(g, ck).astype(jnp.bfloat16)
                )

        order = (1, 2, 0)

        r2_descs = [[None] * 2 for _ in range(3)]
        for g in order:
            d1, d2, d3 = G_DIMS[g]
            for j in range(4):
                r1_descs[g][j].wait_recv()
                ck = r1_chunk(g, j, mybits[d1])
                row = pl.ds(ck * M_OUT, M_OUT)
                P[g][row] = (
                    P[g][row].astype(jnp.float32)
                    + RB1[g][j].astype(jnp.float32)
                ).astype(jnp.bfloat16)
                if j == 1:
                    for sj in range(2):
                        sck = r2_chunk(g, sj, 1 - mybits[d2])
                        r = rdma(g, 4 + sj, sck, RB2[g].at[sj], d2)
                        r.start()
                        r2_descs[g][sj] = r
                        send_descs.append(r)

        r3_descs = [None] * 3
        for g in order:
            d1, d2, d3 = G_DIMS[g]
            for sj in range(2):
                r2_descs[g][sj].wait_recv()
                ck = r2_chunk(g, sj, mybits[d2])
                row = pl.ds(ck * M_OUT, M_OUT)
                P[g][row] = (
                    P[g][row].astype(jnp.float32)
                    + RB2[g][sj].astype(jnp.float32)
                ).astype(jnp.bfloat16)
            sck = cid(flip(mybits, d3))
            r = rdma(g, 6, sck, RB3[g].at[0], d3)
            r.start()
            r3_descs[g] = r
            send_descs.append(r)

        lmax = jnp.float32(0.0)
        for g in order:
            r3_descs[g].wait_recv()
            yg = jnp.maximum(
                P[g][pl.ds(d * M_OUT, M_OUT)].astype(jnp.float32)
                + RB3[g][0].astype(jnp.float32),
                0.0,
            )
            out_ref[:, G_OFF[g]:G_OFF[g] + G_COLS[g]] = yg
            lmax = jnp.maximum(lmax, jnp.max(yg))
        maxsrc_ref[...] = jnp.full((8, 128), lmax, jnp.float32)

        my_pos = _ring(d)
        rdmas = []
        for t in range(1, N_DEV):
            dst = _ring((my_pos + t) % N_DEV)
            r = pltpu.make_async_remote_copy(
                src_ref=maxsrc_ref,
                dst_ref=maxbuf_ref.at[t - 1],
                send_sem=msend_sems.at[t - 1],
                recv_sem=mrecv_sems.at[t - 1],
                device_id=(dst,),
                device_id_type=pl.DeviceIdType.MESH,
            )
            r.start()
            rdmas.append(r)
        for r in send_descs:
            r.wait_send()
        for r in rdmas:
            r.wait_send()
        for r in rdmas:
            r.wait_recv()
        gmax = jnp.maximum(jnp.max(maxbuf_ref[...]), lmax)

        scale = gmax / 127.0
        inv_scale = 127.0 / gmax
        q = jnp.minimum(jnp.round(out_ref[...] * inv_scale), 127.0)
        out_ref[...] = q * scale

    def _dyn_slot(ref, j):
        return ref[j]

    scratch = [
        pltpu.VMEM((8, 128), jnp.float32),
        pltpu.VMEM((N_DEV - 1, 8, 128), jnp.float32),
    ]
    for g in range(3):
        scratch.append(pltpu.VMEM((m, G_COLS[g]), jnp.bfloat16))
    for g in range(3):
        scratch.append(pltpu.VMEM((4, M_OUT, G_COLS[g]), jnp.bfloat16))
    for g in range(3):
        scratch.append(pltpu.VMEM((2, M_OUT, G_COLS[g]), jnp.bfloat16))
    for g in range(3):
        scratch.append(pltpu.VMEM((1, M_OUT, G_COLS[g]), jnp.bfloat16))
    for _ in range(3):
        scratch.append(pltpu.SemaphoreType.DMA((7,)))
    for _ in range(3):
        scratch.append(pltpu.SemaphoreType.DMA((7,)))
    scratch.append(pltpu.SemaphoreType.DMA((N_DEV - 1,)))
    scratch.append(pltpu.SemaphoreType.DMA((N_DEV - 1,)))

    return pl.pallas_call(
        body,
        out_shape=jax.ShapeDtypeStruct((M_OUT, n), jnp.float32),
        in_specs=[
            pl.BlockSpec(memory_space=pltpu.VMEM),
            pl.BlockSpec(memory_space=pltpu.VMEM),
        ],
        out_specs=pl.BlockSpec(memory_space=pltpu.VMEM),
        scratch_shapes=scratch,
        compiler_params=pltpu.CompilerParams(
            collective_id=0, vmem_limit_bytes=64 * 1024 * 1024
        ),
    )(x, w_mat)


# device time: 81059 ns/iter; 1.0068x vs baseline; 1.0068x over previous
import jax
import jax.numpy as jnp
from jax import lax
from jax.experimental import pallas as pl
from jax.experimental.pallas import tpu as pltpu

N_DEV = 8
M_OUT = 512
G_OFF = (0, 768, 1408)
G_COLS = (768, 640, 640)
G_DIMS = ((0, 1, 2), (1, 2, 0), (2, 0, 1))
OTHER = {0: (1, 2), 1: (0, 2), 2: (0, 1)}


def _ring(p):
    return jnp.where(p < 4, p, 11 - p)


def kernel(x, w_mat):
    m, k = x.shape
    _, n = w_mat.shape

    def body(x_ref, w_ref, out_ref, maxsrc_ref, maxbuf_ref, xb_ref, wb_ref,
             p0, p1, p2, rb1_0, rb1_1, rb1_2, rb2_0, rb2_1, rb2_2,
             rb3_0, rb3_1, rb3_2, ss0, ss1, ss2, rs0, rs1, rs2,
             msend_sems, mrecv_sems):
        d = lax.axis_index("i")
        m4 = d % 4
        mybits = [
            jnp.where((m4 == 1) | (m4 == 2), 1, 0),
            jnp.where(m4 >= 2, 1, 0),
            jnp.where(d >= 4, 1, 0),
        ]

        def cid(bits):
            return jnp.where(bits[1] == 0, bits[0], 3 - bits[0]) + 4 * bits[2]

        def flip(bits, dim):
            b = list(bits)
            b[dim] = 1 - b[dim]
            return b

        neighbors = [cid(flip(mybits, dim)) for dim in range(3)]

        barrier_sem = pltpu.get_barrier_semaphore()
        for nbr in neighbors:
            pl.semaphore_signal(
                barrier_sem, inc=1,
                device_id=(nbr,), device_id_type=pl.DeviceIdType.MESH,
            )
        xb_ref[...] = x_ref[...].astype(jnp.bfloat16)
        wb_ref[...] = w_ref[...].astype(jnp.bfloat16)
        pl.semaphore_wait(barrier_sem, 3)

        P = [p0, p1, p2]
        RB1 = [rb1_0, rb1_1, rb1_2]
        RB2 = [rb2_0, rb2_1, rb2_2]
        RB3 = [rb3_0, rb3_1, rb3_2]
        SS = [ss0, ss1, ss2]
        RS = [rs0, rs1, rs2]

        def r1_chunk(g, j, side_bit):
            d1, d2, d3 = G_DIMS[g]
            bits = [None, None, None]
            bits[d1] = side_bit
            bits[d2] = mybits[d2] ^ (1 if j < 2 else 0)
            bits[d3] = mybits[d3] ^ (j & 1)
            return cid(bits)

        def r2_chunk(g, j, side_bit):
            d1, d2, d3 = G_DIMS[g]
            bits = [None, None, None]
            bits[d1] = mybits[d1]
            bits[d2] = side_bit
            bits[d3] = mybits[d3] ^ (1 if j == 0 else 0)
            return cid(bits)

        def dot_chunk(g, ck):
            return jnp.dot(
                xb_ref[pl.ds(ck * M_OUT, M_OUT), :],
                wb_ref[:, G_OFF[g]:G_OFF[g] + G_COLS[g]],
                preferred_element_type=jnp.float32,
            )

        def rdma(g, sem_idx, src_row, dst_ref, dim):
            return pltpu.make_async_remote_copy(
                src_ref=P[g].at[pl.ds(src_row * M_OUT, M_OUT)],
                dst_ref=dst_ref,
                send_sem=SS[g].at[sem_idx],
                recv_sem=RS[g].at[sem_idx],
                device_id=(neighbors[dim],),
                device_id_type=pl.DeviceIdType.MESH,
            )

        send_descs = []

        r1_descs = [[None] * 4 for _ in range(3)]
        for j in range(4):
            for g in range(3):
                dim = G_DIMS[g][0]
                ck = r1_chunk(g, j, 1 - mybits[dim])
                P[g][pl.ds(ck * M_OUT, M_OUT)] = (
                    dot_chunk(g, ck).astype(jnp.bfloat16)
                )
                r = rdma(g, j, ck, RB1[g].at[j], dim)
                r.start()
                r1_descs[g][j] = r
                send_descs.append(r)

        for j in range(4):
            for g in range(3):
                dim = G_DIMS[g][0]
                ck = r1_chunk(g, j, mybits[dim])
                P[g][pl.ds(ck * M_OUT, M_OUT)] = (
                    dot_chunk(g, ck).astype(jnp.bfloat16)
                )

        order = (1, 2, 0)

        r2_descs = [[None] * 2 for _ in range(3)]
        for g in order:
            d1, d2, d3 = G_DIMS[g]
            for j in range(4):
                r1_descs[g][j].wait_recv()
                ck = r1_chunk(g, j, mybits[d1])
                row = pl.ds(ck * M_OUT, M_OUT)
                P[g][row] = (
                    P[g][row].astype(jnp.float32)
                    + RB1[g][j].astype(jnp.float32)
                ).astype(jnp.bfloat16)
                if j == 1:
                    for sj in range(2):
                        sck = r2_chunk(g, sj, 1 - mybits[d2])
                        r = rdma(g, 4 + sj, sck, RB2[g].at[sj], d2)
                        r.start()
                        r2_descs[g][sj] = r
                        send_descs.append(r)

        r3_descs = [None] * 3
        for g in order:
            d1, d2, d3 = G_DIMS[g]
            for sj in range(2):
                r2_descs[g][sj].wait_recv()
                ck = r2_chunk(g, sj, mybits[d2])
                row = pl.ds(ck * M_OUT, M_OUT)
                P[g][row] = (
                    P[g][row].astype(jnp.float32)
                    + RB2[g][sj].astype(jnp.float32)
                ).astype(jnp.bfloat16)
            sck = cid(flip(mybits, d3))
            r = rdma(g, 6, sck, RB3[g].at[0], d3)
            r.start()
            r3_descs[g] = r
            send_descs.append(r)

        lmax = jnp.float32(0.0)
        for g in order:
            r3_descs[g].wait_recv()
            yg = jnp.maximum(
                P[g][pl.ds(d * M_OUT, M_OUT)].astype(jnp.float32)
                + RB3[g][0].astype(jnp.float32),
                0.0,
            )
            out_ref[:, G_OFF[g]:G_OFF[g] + G_COLS[g]] = yg
            lmax = jnp.maximum(lmax, jnp.max(yg))
        maxsrc_ref[...] = jnp.full((8, 128), lmax, jnp.float32)

        my_pos = _ring(d)
        rdmas = []
        for t in range(1, N_DEV):
            dst = _ring((my_pos + t) % N_DEV)
            r = pltpu.make_async_remote_copy(
                src_ref=maxsrc_ref,
                dst_ref=maxbuf_ref.at[t - 1],
                send_sem=msend_sems.at[t - 1],
                recv_sem=mrecv_sems.at[t - 1],
                device_id=(dst,),
                device_id_type=pl.DeviceIdType.MESH,
            )
            r.start()
            rdmas.append(r)
        for r in send_descs:
            r.wait_send()
        for r in rdmas:
            r.wait_send()
        for r in rdmas:
            r.wait_recv()
        gmax = jnp.maximum(jnp.max(maxbuf_ref[...]), lmax)

        scale = gmax / 127.0
        inv_scale = 127.0 / gmax
        q = jnp.minimum(jnp.round(out_ref[...] * inv_scale), 127.0)
        out_ref[...] = q * scale

    def _dyn_slot(ref, j):
        return ref[j]

    scratch = [
        pltpu.VMEM((8, 128), jnp.float32),
        pltpu.VMEM((N_DEV - 1, 8, 128), jnp.float32),
        pltpu.VMEM((m, k), jnp.bfloat16),
        pltpu.VMEM((k, n), jnp.bfloat16),
    ]
    for g in range(3):
        scratch.append(pltpu.VMEM((m, G_COLS[g]), jnp.bfloat16))
    for g in range(3):
        scratch.append(pltpu.VMEM((4, M_OUT, G_COLS[g]), jnp.bfloat16))
    for g in range(3):
        scratch.append(pltpu.VMEM((2, M_OUT, G_COLS[g]), jnp.bfloat16))
    for g in range(3):
        scratch.append(pltpu.VMEM((1, M_OUT, G_COLS[g]), jnp.bfloat16))
    for _ in range(3):
        scratch.append(pltpu.SemaphoreType.DMA((7,)))
    for _ in range(3):
        scratch.append(pltpu.SemaphoreType.DMA((7,)))
    scratch.append(pltpu.SemaphoreType.DMA((N_DEV - 1,)))
    scratch.append(pltpu.SemaphoreType.DMA((N_DEV - 1,)))

    return pl.pallas_call(
        body,
        out_shape=jax.ShapeDtypeStruct((M_OUT, n), jnp.float32),
        in_specs=[
            pl.BlockSpec(memory_space=pltpu.VMEM),
            pl.BlockSpec(memory_space=pltpu.VMEM),
        ],
        out_specs=pl.BlockSpec(memory_space=pltpu.VMEM),
        scratch_shapes=scratch,
        compiler_params=pltpu.CompilerParams(
            collective_id=0, vmem_limit_bytes=64 * 1024 * 1024
        ),
    )(x, w_mat)


# device time: 76032 ns/iter; 1.0734x vs baseline; 1.0661x over previous
import jax
import jax.numpy as jnp
from jax import lax
from jax.experimental import pallas as pl
from jax.experimental.pallas import tpu as pltpu

N_DEV = 8
M_OUT = 512
NG = 6
G_COLS = (256, 512, 384, 256, 128, 512)
G_OFF = (0, 256, 768, 1152, 1408, 1536)
G_DIMS = (
    (0, 1, 2), (0, 2, 1), (1, 0, 2), (1, 2, 0), (2, 0, 1), (2, 1, 0),
)
ORDER_R1P1 = (4, 0, 2, 3, 5, 1)
ORDER_R1P2 = (4, 2, 0, 3, 5, 1)
ORDER_R2P1 = (0, 3, 4, 5, 1, 2)
ORDER_R2P2 = (0, 3, 4, 5, 1, 2)
ORDER_R3 = (4, 3, 0, 5, 1, 2)


def _ring(p):
    return jnp.where(p < 4, p, 11 - p)


def kernel(x, w_mat):
    m, k = x.shape
    _, n = w_mat.shape

    def body(x_ref, w_ref, out_ref, maxsrc_ref, maxbuf_ref, xb_ref, wb_ref,
             *scr):
        P = scr[0:NG]
        RB1 = scr[NG:2 * NG]
        RB2 = scr[2 * NG:3 * NG]
        RB3 = scr[3 * NG:4 * NG]
        SS = scr[4 * NG:5 * NG]
        RS = scr[5 * NG:6 * NG]
        msend_sems = scr[6 * NG]
        mrecv_sems = scr[6 * NG + 1]

        d = lax.axis_index("i")
        m4 = d % 4
        mybits = [
            jnp.where((m4 == 1) | (m4 == 2), 1, 0),
            jnp.where(m4 >= 2, 1, 0),
            jnp.where(d >= 4, 1, 0),
        ]

        def cid(bits):
            return jnp.where(bits[1] == 0, bits[0], 3 - bits[0]) + 4 * bits[2]

        def flip(bits, dim):
            b = list(bits)
            b[dim] = 1 - b[dim]
            return b

        neighbors = [cid(flip(mybits, dim)) for dim in range(3)]

        barrier_sem = pltpu.get_barrier_semaphore()
        for nbr in neighbors:
            pl.semaphore_signal(
                barrier_sem, inc=1,
                device_id=(nbr,), device_id_type=pl.DeviceIdType.MESH,
            )
        xb_ref[...] = x_ref[...].astype(jnp.bfloat16)
        wb_ref[...] = w_ref[...].astype(jnp.bfloat16)
        pl.semaphore_wait(barrier_sem, 3)

        def r1_chunk(g, j, side_bit):
            d1, d2, d3 = G_DIMS[g]
            bits = [None, None, None]
            bits[d1] = side_bit
            bits[d2] = mybits[d2] ^ (1 if j < 2 else 0)
            bits[d3] = mybits[d3] ^ (j & 1)
            return cid(bits)

        def r2_chunk(g, j, side_bit):
            d1, d2, d3 = G_DIMS[g]
            bits = [None, None, None]
            bits[d1] = mybits[d1]
            bits[d2] = side_bit
            bits[d3] = mybits[d3] ^ (1 if j == 0 else 0)
            return cid(bits)

        def dot_chunk(g, ck):
            return jnp.dot(
                xb_ref[pl.ds(ck * M_OUT, M_OUT), :],
                wb_ref[:, G_OFF[g]:G_OFF[g] + G_COLS[g]],
                preferred_element_type=jnp.float32,
            )

        def rdma(g, sem_idx, src_row, dst_ref, dim):
            return pltpu.make_async_remote_copy(
                src_ref=P[g].at[pl.ds(src_row * M_OUT, M_OUT)],
                dst_ref=dst_ref,
                send_sem=SS[g].at[sem_idx],
                recv_sem=RS[g].at[sem_idx],
                device_id=(neighbors[dim],),
                device_id_type=pl.DeviceIdType.MESH,
            )

        send_descs = []

        r1_descs = [[None] * 4 for _ in range(NG)]
        for j in range(4):
            for g in range(NG):
                dim = G_DIMS[g][0]
                ck = r1_chunk(g, j, 1 - mybits[dim])
                P[g][pl.ds(ck * M_OUT, M_OUT)] = (
                    dot_chunk(g, ck).astype(jnp.bfloat16)
                )
                r = rdma(g, j, ck, RB1[g].at[j], dim)
                r.start()
                r1_descs[g][j] = r
                send_descs.append(r)

        for j in range(4):
            for g in range(NG):
                dim = G_DIMS[g][0]
                ck = r1_chunk(g, j, mybits[dim])
                P[g][pl.ds(ck * M_OUT, M_OUT)] = (
                    dot_chunk(g, ck).astype(jnp.bfloat16)
                )

        r2_descs = [[None] * 2 for _ in range(NG)]

        def r1_add(g, j):
            ck = r1_chunk(g, j, mybits[G_DIMS[g][0]])
            row = pl.ds(ck * M_OUT, M_OUT)
            r1_descs[g][j].wait_recv()
            P[g][row] = (
                P[g][row].astype(jnp.float32)
                + RB1[g][j].astype(jnp.float32)
            ).astype(jnp.bfloat16)

        for g in ORDER_R1P1:
            d2 = G_DIMS[g][1]
            for j in range(2):
                r1_add(g, j)
            for sj in range(2):
                sck = r2_chunk(g, sj, 1 - mybits[d2])
                r = rdma(g, 4 + sj, sck, RB2[g].at[sj], d2)
                r.start()
                r2_descs[g][sj] = r
                send_descs.append(r)

        for g in ORDER_R1P2:
            for j in range(2, 4):
                r1_add(g, j)

        def r2_add(g, sj):
            ck = r2_chunk(g, sj, mybits[G_DIMS[g][1]])
            row = pl.ds(ck * M_OUT, M_OUT)
            r2_descs[g][sj].wait_recv()
            P[g][row] = (
                P[g][row].astype(jnp.float32)
                + RB2[g][sj].astype(jnp.float32)
            ).astype(jnp.bfloat16)

        r3_descs = [None] * NG
        for g in ORDER_R2P1:
            d3 = G_DIMS[g][2]
            r2_add(g, 0)
            sck = cid(flip(mybits, d3))
            r = rdma(g, 6, sck, RB3[g].at[0], d3)
            r.start()
            r3_descs[g] = r
            send_descs.append(r)

        for g in ORDER_R2P2:
            r2_add(g, 1)

        lmax = jnp.float32(0.0)
        for g in ORDER_R3:
            r3_descs[g].wait_recv()
            yg = jnp.maximum(
                P[g][pl.ds(d * M_OUT, M_OUT)].astype(jnp.float32)
                + RB3[g][0].astype(jnp.float32),
                0.0,
            )
            out_ref[:, G_OFF[g]:G_OFF[g] + G_COLS[g]] = yg
            lmax = jnp.maximum(lmax, jnp.max(yg))
        maxsrc_ref[...] = jnp.full((8, 128), lmax, jnp.float32)

        my_pos = _ring(d)
        rdmas = []
        for t in range(1, N_DEV):
            dst = _ring((my_pos + t) % N_DEV)
            r = pltpu.make_async_remote_copy(
                src_ref=maxsrc_ref,
                dst_ref=maxbuf_ref.at[t - 1],
                send_sem=msend_sems.at[t - 1],
                recv_sem=mrecv_sems.at[t - 1],
                device_id=(dst,),
                device_id_type=pl.DeviceIdType.MESH,
            )
            r.start()
            rdmas.append(r)
        for r in send_descs:
            r.wait_send()
        for r in rdmas:
            r.wait_send()
        for r in rdmas:
            r.wait_recv()
        gmax = jnp.maximum(jnp.max(maxbuf_ref[...]), lmax)

        scale = gmax / 127.0
        inv_scale = 127.0 / gmax
        q = jnp.minimum(jnp.round(out_ref[...] * inv_scale), 127.0)
        out_ref[...] = q * scale

    scratch = [
        pltpu.VMEM((8, 128), jnp.float32),
        pltpu.VMEM((N_DEV - 1, 8, 128), jnp.float32),
        pltpu.VMEM((m, k), jnp.bfloat16),
        pltpu.VMEM((k, n), jnp.bfloat16),
    ]
    for g in range(NG):
        scratch.append(pltpu.VMEM((m, G_COLS[g]), jnp.bfloat16))
    for g in range(NG):
        scratch.append(pltpu.VMEM((4, M_OUT, G_COLS[g]), jnp.bfloat16))
    for g in range(NG):
        scratch.append(pltpu.VMEM((2, M_OUT, G_COLS[g]), jnp.bfloat16))
    for g in range(NG):
        scratch.append(pltpu.VMEM((1, M_OUT, G_COLS[g]), jnp.bfloat16))
    for _ in range(NG):
        scratch.append(pltpu.SemaphoreType.DMA((7,)))
    for _ in range(NG):
        scratch.append(pltpu.SemaphoreType.DMA((7,)))
    scratch.append(pltpu.SemaphoreType.DMA((N_DEV - 1,)))
    scratch.append(pltpu.SemaphoreType.DMA((N_DEV - 1,)))

    return pl.pallas_call(
        body,
        out_shape=jax.ShapeDtypeStruct((M_OUT, n), jnp.float32),
        in_specs=[
            pl.BlockSpec(memory_space=pltpu.VMEM),
            pl.BlockSpec(memory_space=pltpu.VMEM),
        ],
        out_specs=pl.BlockSpec(memory_space=pltpu.VMEM),
        scratch_shapes=scratch,
        compiler_params=pltpu.CompilerParams(
            collective_id=0, vmem_limit_bytes=64 * 1024 * 1024
        ),
    )(x, w_mat)
